# baseline (speedup 1.0000x reference)
import functools

import jax
import jax.numpy as jnp
from jax import lax
from jax.experimental import pallas as pl
from jax.experimental.pallas import tpu as pltpu
from jax.experimental.pallas import tpu_sc as plsc

K = 8
_BR = 512
_BM = 256


def _topk_body(nprompt, br, q_ref, k_ref, vals_ref, idxs_ref, scores_ref,
               p_ref, l_ref, i_ref, il_ref):
    half = nprompt // 2
    c = lax.dot_general(q_ref[...], k_ref[...], (((1,), (1,)), ((), ())),
                        preferred_element_type=jnp.float32)
    d = (1.0 - c) * 0.5
    da = d[:, :half]
    db = d[:, half:]
    colf = lax.broadcasted_iota(jnp.int32, (br, half), 1).astype(jnp.float32)
    bwin = db < da
    p_ref[...] = jnp.where(bwin, db, da)
    l_ref[...] = jnp.where(bwin, da, db)
    i_ref[...] = jnp.where(bwin, colf + half, colf)
    il_ref[...] = jnp.where(bwin, colf, colf + half)
    m = jnp.min(p_ref[...], axis=1)
    for r in range(K):
        p = p_ref[...]
        ivals = i_ref[...]
        am = jnp.min(jnp.where(p == m[:, None], ivals, float(nprompt)), axis=1)
        hit = ivals == am[:, None]
        pnew = jnp.where(hit, l_ref[...], p)
        p_ref[...] = pnew
        i_ref[...] = jnp.where(hit, il_ref[...], ivals)
        l_ref[...] = jnp.where(hit, 3.0, l_ref[...])
        vals_ref[r, :] = m
        idxs_ref[r, :] = am.astype(jnp.int32)
        if r < K - 1:
            m = jnp.min(pnew, axis=1)
    v = vals_ref[...]
    z = 1.0 - v
    z = z - jnp.max(z, axis=0, keepdims=True)
    e = jnp.exp(z)
    scores_ref[...] = e / jnp.sum(e, axis=0, keepdims=True) / K


def _topk_call(qn, kn):
    b, qd = qn.shape
    np_, _ = kn.shape
    return pl.pallas_call(
        functools.partial(_topk_body, np_, _BR),
        grid=(b // _BR,),
        in_specs=[pl.BlockSpec((_BR, qd), lambda i: (i, 0)),
                  pl.BlockSpec((np_, qd), lambda i: (0, 0))],
        out_specs=[pl.BlockSpec((K, _BR), lambda i: (0, i)),
                   pl.BlockSpec((K, _BR), lambda i: (0, i)),
                   pl.BlockSpec((K, _BR), lambda i: (0, i))],
        out_shape=[jax.ShapeDtypeStruct((K, b), jnp.float32),
                   jax.ShapeDtypeStruct((K, b), jnp.int32),
                   jax.ShapeDtypeStruct((K, b), jnp.float32)],
        scratch_shapes=[pltpu.VMEM((_BR, np_ // 2), jnp.float32),
                        pltpu.VMEM((_BR, np_ // 2), jnp.float32),
                        pltpu.VMEM((_BR, np_ // 2), jnp.float32),
                        pltpu.VMEM((_BR, np_ // 2), jnp.float32)],
    )(qn, kn)


def _sc_gather_avg(table2, gidx, loff, w, b, pd):
    n = gidx.shape[0]
    wrow = table2.shape[1]
    info = plsc.get_sparse_core_info()
    nw = info.num_cores * info.num_subcores
    rpw = n // nw
    ch = 128
    nch = rpw // ch
    qch = ch // K
    qpw = rpw // K
    nv = pd // 16

    mesh = plsc.VectorSubcoreMesh(core_axis_name="c", subcore_axis_name="s")

    @functools.partial(
        pl.kernel, mesh=mesh,
        out_type=jax.ShapeDtypeStruct((b, pd), jnp.float32),
        scratch_types=[pltpu.VMEM((rpw,), jnp.int32),
                       pltpu.VMEM((rpw,), jnp.int32),
                       pltpu.VMEM((rpw,), jnp.float32),
                       pltpu.VMEM((2, ch, wrow), jnp.float32),
                       pltpu.VMEM((qpw, pd), jnp.float32),
                       pltpu.SemaphoreType.DMA,
                       pltpu.SemaphoreType.DMA],
    )
    def gather_kernel(tab_hbm, gidx_hbm, loff_hbm, w_hbm, out_hbm,
                      gidx_v, loff_v, w_v, rows_v, acc_v, sem0, sem1):
        wid = lax.axis_index("s") * info.num_cores + lax.axis_index("c")
        base = wid * rpw
        pltpu.sync_copy(gidx_hbm.at[pl.ds(base, rpw)], gidx_v)
        pltpu.sync_copy(loff_hbm.at[pl.ds(base, rpw)], loff_v)
        pltpu.sync_copy(w_hbm.at[pl.ds(base, rpw)], w_v)
        sems = (sem0, sem1)
        handles = [None, None]
        handles[0] = pltpu.async_copy(
            tab_hbm.at[gidx_v.at[pl.ds(0, ch)]], rows_v.at[0], sems[0])

        for c in range(nch):
            s = c % 2
            if c + 1 < nch:
                handles[(c + 1) % 2] = pltpu.async_copy(
                    tab_hbm.at[gidx_v.at[pl.ds((c + 1) * ch, ch)]],
                    rows_v.at[(c + 1) % 2], sems[(c + 1) % 2])
            handles[s].wait()

            def q_body(qp, _):
                r0 = c * ch + qp * 16
                w16 = w_v[pl.ds(r0, 16)]
                o16 = loff_v[pl.ds(r0, 16)]
                for half in range(2):
                    q = qp * 2 + half
                    acc = [jnp.zeros((16,), jnp.float32) for _ in range(nv)]
                    for k in range(K):
                        i = half * K + k
                        wk = w16[i]
                        ok = o16[i]
                        for v in range(nv):
                            acc[v] = acc[v] + wk * rows_v[
                                s, q * K + k, pl.ds(ok + v * 16, 16)]
                    for v in range(nv):
                        acc_v[c * qch + q, pl.ds(v * 16, 16)] = acc[v]
                return _

            lax.fori_loop(0, qch // 2, q_body, None)

        pltpu.sync_copy(acc_v, out_hbm.at[pl.ds(wid * qpw, qpw)])

    return gather_kernel(table2, gidx, loff, w)


def _mlp_body(s_, br, avg_ref, w1_ref, b1_ref, w2_ref, b2_ref, x_ref):
    h = lax.dot_general(avg_ref[...], w1_ref[...], (((1,), (1,)), ((), ())),
                        preferred_element_type=jnp.float32) + b1_ref[...]
    h = jnp.maximum(h, 0.0)
    o = jnp.sum(h * w2_ref[...], axis=1, keepdims=True) + b2_ref[...]
    x_ref[...] = jnp.broadcast_to(o, (br, s_))


def _mlp_call(avg, W1, b1, W2, b2, s_):
    b, pd = avg.shape
    hd = W1.shape[0]
    return pl.pallas_call(
        functools.partial(_mlp_body, s_, _BM),
        grid=(b // _BM,),
        in_specs=[pl.BlockSpec((_BM, pd), lambda i: (i, 0)),
                  pl.BlockSpec((hd, pd), lambda i: (0, 0)),
                  pl.BlockSpec((1, hd), lambda i: (0, 0)),
                  pl.BlockSpec((W2.shape[0], hd), lambda i: (0, 0)),
                  pl.BlockSpec((1, 1), lambda i: (0, 0))],
        out_specs=pl.BlockSpec((_BM, s_), lambda i: (i, 0)),
        out_shape=jax.ShapeDtypeStruct((b, s_), jnp.float32),
    )(avg, W1, b1, W2, b2)


def kernel(query, cloud, keys, prompts, W1, b1, W2, b2):
    s_ = cloud.shape[1]
    b = query.shape[0]
    pd = prompts.shape[1]
    qn = query / jnp.maximum(
        jnp.linalg.norm(query, axis=-1, keepdims=True), 1e-12)
    kn = keys / jnp.maximum(
        jnp.linalg.norm(keys, axis=-1, keepdims=True), 1e-12)
    vals_kb, idxs_kb, scores_kb = _topk_call(qn, kn)
    vals = vals_kb.T
    idxs = idxs_kb.T.reshape(-1)
    w = scores_kb.T.reshape(-1)
    table2 = prompts.reshape(prompts.shape[0] // 2, 2 * pd)
    gidx = lax.shift_right_logical(idxs, 1)
    loff = (idxs & 1) * pd
    avg = _sc_gather_avg(table2, gidx, loff, w, b, pd)
    x2 = _mlp_call(avg, W1, b1.reshape(1, -1), W2, b2.reshape(1, 1), s_)
    return (x2[..., None], vals)


# speedup vs baseline: 1.0339x; 1.0339x over previous
import functools

import jax
import jax.numpy as jnp
from jax import lax
from jax.experimental import pallas as pl
from jax.experimental.pallas import tpu as pltpu
from jax.experimental.pallas import tpu_sc as plsc

K = 8
_BR = 128
_BM = 256


def _topk_body(nprompt, br, q_ref, k_ref, vals_ref, idxs_ref, scores_ref,
               p_ref, l_ref, i_ref, il_ref):
    half = nprompt // 2
    c = lax.dot_general(q_ref[...], k_ref[...], (((1,), (1,)), ((), ())),
                        preferred_element_type=jnp.float32)
    d = (1.0 - c) * 0.5
    da = d[:, :half]
    db = d[:, half:]
    colf = lax.broadcasted_iota(jnp.int32, (br, half), 1).astype(jnp.float32)
    bwin = db < da
    p_ref[...] = jnp.where(bwin, db, da)
    l_ref[...] = jnp.where(bwin, da, db)
    i_ref[...] = jnp.where(bwin, colf + half, colf)
    il_ref[...] = jnp.where(bwin, colf, colf + half)
    m = jnp.min(p_ref[...], axis=1)
    for r in range(K):
        p = p_ref[...]
        ivals = i_ref[...]
        am = jnp.min(jnp.where(p == m[:, None], ivals, float(nprompt)), axis=1)
        hit = ivals == am[:, None]
        pnew = jnp.where(hit, l_ref[...], p)
        p_ref[...] = pnew
        i_ref[...] = jnp.where(hit, il_ref[...], ivals)
        l_ref[...] = jnp.where(hit, 3.0, l_ref[...])
        vals_ref[r, :] = m
        idxs_ref[r, :] = am.astype(jnp.int32)
        if r < K - 1:
            m = jnp.min(pnew, axis=1)
    v = vals_ref[...]
    z = 1.0 - v
    z = z - jnp.max(z, axis=0, keepdims=True)
    e = jnp.exp(z)
    scores_ref[...] = e / jnp.sum(e, axis=0, keepdims=True) / K


def _topk_call(qn, kn):
    b, qd = qn.shape
    np_, _ = kn.shape
    return pl.pallas_call(
        functools.partial(_topk_body, np_, _BR),
        grid=(b // _BR,),
        in_specs=[pl.BlockSpec((_BR, qd), lambda i: (i, 0)),
                  pl.BlockSpec((np_, qd), lambda i: (0, 0))],
        out_specs=[pl.BlockSpec((K, _BR), lambda i: (0, i)),
                   pl.BlockSpec((K, _BR), lambda i: (0, i)),
                   pl.BlockSpec((K, _BR), lambda i: (0, i))],
        out_shape=[jax.ShapeDtypeStruct((K, b), jnp.float32),
                   jax.ShapeDtypeStruct((K, b), jnp.int32),
                   jax.ShapeDtypeStruct((K, b), jnp.float32)],
        scratch_shapes=[pltpu.VMEM((_BR, np_ // 2), jnp.float32),
                        pltpu.VMEM((_BR, np_ // 2), jnp.float32),
                        pltpu.VMEM((_BR, np_ // 2), jnp.float32),
                        pltpu.VMEM((_BR, np_ // 2), jnp.float32)],
    )(qn, kn)


def _sc_gather_avg(table2, gidx, loff, w, b, pd):
    n = gidx.shape[0]
    wrow = table2.shape[1]
    info = plsc.get_sparse_core_info()
    nw = info.num_cores * info.num_subcores
    rpw = n // nw
    ch = 128
    nch = rpw // ch
    qch = ch // K
    qpw = rpw // K
    nv = pd // 16

    mesh = plsc.VectorSubcoreMesh(core_axis_name="c", subcore_axis_name="s")

    @functools.partial(
        pl.kernel, mesh=mesh,
        out_type=jax.ShapeDtypeStruct((b, pd), jnp.float32),
        scratch_types=[pltpu.VMEM((rpw,), jnp.int32),
                       pltpu.VMEM((rpw,), jnp.int32),
                       pltpu.VMEM((rpw,), jnp.float32),
                       pltpu.VMEM((2, ch, wrow), jnp.float32),
                       pltpu.VMEM((qpw, pd), jnp.float32),
                       pltpu.SemaphoreType.DMA,
                       pltpu.SemaphoreType.DMA],
    )
    def gather_kernel(tab_hbm, gidx_hbm, loff_hbm, w_hbm, out_hbm,
                      gidx_v, loff_v, w_v, rows_v, acc_v, sem0, sem1):
        wid = lax.axis_index("s") * info.num_cores + lax.axis_index("c")
        base = wid * rpw
        pltpu.sync_copy(gidx_hbm.at[pl.ds(base, rpw)], gidx_v)
        pltpu.sync_copy(loff_hbm.at[pl.ds(base, rpw)], loff_v)
        pltpu.sync_copy(w_hbm.at[pl.ds(base, rpw)], w_v)
        sems = (sem0, sem1)
        handles = [None, None]
        handles[0] = pltpu.async_copy(
            tab_hbm.at[gidx_v.at[pl.ds(0, ch)]], rows_v.at[0], sems[0])

        for c in range(nch):
            s = c % 2
            if c + 1 < nch:
                handles[(c + 1) % 2] = pltpu.async_copy(
                    tab_hbm.at[gidx_v.at[pl.ds((c + 1) * ch, ch)]],
                    rows_v.at[(c + 1) % 2], sems[(c + 1) % 2])
            handles[s].wait()

            def q_body(qp, _):
                r0 = c * ch + qp * 16
                w16 = w_v[pl.ds(r0, 16)]
                o16 = loff_v[pl.ds(r0, 16)]
                for half in range(2):
                    q = qp * 2 + half
                    acc = [jnp.zeros((16,), jnp.float32) for _ in range(nv)]
                    for k in range(K):
                        i = half * K + k
                        wk = w16[i]
                        ok = o16[i]
                        for v in range(nv):
                            acc[v] = acc[v] + wk * rows_v[
                                s, q * K + k, pl.ds(ok + v * 16, 16)]
                    for v in range(nv):
                        acc_v[c * qch + q, pl.ds(v * 16, 16)] = acc[v]
                return _

            lax.fori_loop(0, qch // 2, q_body, None)

        pltpu.sync_copy(acc_v, out_hbm.at[pl.ds(wid * qpw, qpw)])

    return gather_kernel(table2, gidx, loff, w)


def _mlp_body(s_, br, avg_ref, w1_ref, b1_ref, w2_ref, b2_ref, x_ref):
    h = lax.dot_general(avg_ref[...], w1_ref[...], (((1,), (1,)), ((), ())),
                        preferred_element_type=jnp.float32) + b1_ref[...]
    h = jnp.maximum(h, 0.0)
    o = jnp.sum(h * w2_ref[...], axis=1, keepdims=True) + b2_ref[...]
    x_ref[...] = jnp.broadcast_to(o, (br, s_))


def _mlp_call(avg, W1, b1, W2, b2, s_):
    b, pd = avg.shape
    hd = W1.shape[0]
    return pl.pallas_call(
        functools.partial(_mlp_body, s_, _BM),
        grid=(b // _BM,),
        in_specs=[pl.BlockSpec((_BM, pd), lambda i: (i, 0)),
                  pl.BlockSpec((hd, pd), lambda i: (0, 0)),
                  pl.BlockSpec((1, hd), lambda i: (0, 0)),
                  pl.BlockSpec((W2.shape[0], hd), lambda i: (0, 0)),
                  pl.BlockSpec((1, 1), lambda i: (0, 0))],
        out_specs=pl.BlockSpec((_BM, s_), lambda i: (i, 0)),
        out_shape=jax.ShapeDtypeStruct((b, s_), jnp.float32),
    )(avg, W1, b1, W2, b2)


def kernel(query, cloud, keys, prompts, W1, b1, W2, b2):
    s_ = cloud.shape[1]
    b = query.shape[0]
    pd = prompts.shape[1]
    qn = query / jnp.maximum(
        jnp.linalg.norm(query, axis=-1, keepdims=True), 1e-12)
    kn = keys / jnp.maximum(
        jnp.linalg.norm(keys, axis=-1, keepdims=True), 1e-12)
    vals_kb, idxs_kb, scores_kb = _topk_call(qn, kn)
    vals = vals_kb.T
    idxs = idxs_kb.T.reshape(-1)
    w = scores_kb.T.reshape(-1)
    table2 = prompts.reshape(prompts.shape[0] // 2, 2 * pd)
    gidx = lax.shift_right_logical(idxs, 1)
    loff = (idxs & 1) * pd
    avg = _sc_gather_avg(table2, gidx, loff, w, b, pd)
    x2 = _mlp_call(avg, W1, b1.reshape(1, -1), W2, b2.reshape(1, 1), s_)
    return (x2[..., None], vals)
